# baseline (speedup 1.0000x reference)
import functools

import jax
import jax.numpy as jnp
from jax import lax
from jax.experimental import pallas as pl
from jax.experimental.pallas import tpu as pltpu
from jax.experimental.pallas import tpu_sc as plsc

DIM = 64
BATCH = 4096
HIST = 50
NC, NS = 2, 16
NW = NC * NS
RPW = BATCH // NW
NBUF = 4
LAG = 2
NG = RPW // NBUF

_mesh = plsc.VectorSubcoreMesh(core_axis_name="c", subcore_axis_name="s")


@functools.partial(
    pl.kernel,
    mesh=_mesh,
    out_type=jax.ShapeDtypeStruct((BATCH, HIST, DIM), jnp.float32),
    scratch_types=[
        pltpu.VMEM((RPW, HIST), jnp.int32),
        pltpu.VMEM((NBUF, HIST, DIM), jnp.float32),
        [pltpu.SemaphoreType.DMA] * NBUF,
        [pltpu.SemaphoreType.DMA] * NBUF,
    ],
    compiler_params=pltpu.CompilerParams(use_tc_tiling_on_sc=False),
)
def _emb_gather(inp_hbm, table_hbm, out_hbm, idx_v, rows_v, gsems, osems):
    wid = lax.axis_index("s") * NC + lax.axis_index("c")
    row0 = wid * RPW
    pltpu.sync_copy(inp_hbm.at[pl.ds(row0, RPW), :], idx_v)

    def gather_start(r, b):
        pltpu.async_copy(table_hbm.at[idx_v.at[r]], rows_v.at[b], gsems[b])

    def gather_wait(r, b):
        pltpu.make_async_copy(
            table_hbm.at[idx_v.at[r]], rows_v.at[b], gsems[b]).wait()

    def out_start(r, b):
        pltpu.async_copy(rows_v.at[b], out_hbm.at[row0 + r], osems[b])

    def out_wait(r, b):
        pltpu.make_async_copy(
            rows_v.at[b], out_hbm.at[row0 + r], osems[b]).wait()

    for b in range(NBUF):
        gather_start(b, b)

    def group(g, carry):
        for b in range(NBUF):
            r = g * NBUF + b
            gather_wait(r, b)
            out_start(r, b)
            bp = (b - LAG) % NBUF
            rp = r - LAG
            rn = rp + NBUF

            @pl.when(rp >= 0)
            def _():
                out_wait(rp, bp)

                @pl.when(rn < RPW)
                def _():
                    gather_start(rn, bp)

        return carry

    lax.fori_loop(0, NG, group, 0)

    for r in range(RPW - LAG, RPW):
        out_wait(r, r % NBUF)


def kernel(inp, table):
    return _emb_gather(inp, table)
